# baseline (speedup 1.0000x reference)
import functools

import jax
import jax.numpy as jnp
from jax import lax
from jax.experimental import pallas as pl
from jax.experimental.pallas import tpu as pltpu
from jax.experimental.pallas import tpu_sc as plsc

_BATCH = 16384
_D = 64
_NC = 2
_NS = 16
_NW = _NC * _NS
_BPW = _BATCH // _NW
_BLK = 2048


@functools.partial(
    pl.kernel,
    out_type=[
        jax.ShapeDtypeStruct((_BATCH, 2 * _D), jnp.float32),
        jax.ShapeDtypeStruct((_BATCH, 2 * _D), jnp.float32),
    ],
    mesh=plsc.VectorSubcoreMesh(core_axis_name="c", subcore_axis_name="s"),
    scratch_types=[
        pltpu.VMEM((_BPW,), jnp.int32),
        pltpu.VMEM((_BPW, 2 * _D), jnp.float32),
        pltpu.SemaphoreType.DMA,
    ],
)
def _sc_gather(uidx_hbm, iidx_hbm, uemb_hbm, iemb_hbm, uout_hbm, iout_hbm,
               idx_v, rows_v, sem):
    wid = lax.axis_index("s") * _NC + lax.axis_index("c")
    base = wid * _BPW

    def one_table(idx_hbm, emb_hbm, out_hbm):
        pltpu.sync_copy(idx_hbm.at[pl.ds(base, _BPW)], idx_v)
        pltpu.async_copy(emb_hbm.at[idx_v], rows_v, sem).wait()
        pltpu.sync_copy(rows_v, out_hbm.at[pl.ds(base, _BPW)])

    one_table(uidx_hbm, uemb_hbm, uout_hbm)
    one_table(iidx_hbm, iemb_hbm, iout_hbm)


def _mlp_body(u_ref, i_ref, ub_ref, ib_ref, w1u_ref, w1i_ref, b1_ref,
              w2_ref, b2_ref, w3_ref, b3_ref, out_ref):
    u = jnp.where(ub_ref[...] == 1, u_ref[:, _D:], u_ref[:, :_D])
    i = jnp.where(ib_ref[...] == 1, i_ref[:, _D:], i_ref[:, :_D])
    dot = functools.partial(jnp.dot, preferred_element_type=jnp.float32,
                            precision=lax.Precision.HIGHEST)
    h = dot(u, w1u_ref[...]) + dot(i, w1i_ref[...])
    h = jnp.maximum(h + b1_ref[...], 0.0)
    h = jnp.maximum(dot(h, w2_ref[...]) + b2_ref[...], 0.0)
    s = jnp.sum(h * w3_ref[...], axis=1, keepdims=True) + b3_ref[...]
    out_ref[...] = jax.nn.sigmoid(s)


def _mlp(uo, io, ub, ib, w1u, w1i, b1, w2, b2, w3, b3):
    return pl.pallas_call(
        _mlp_body,
        grid=(_BATCH // _BLK,),
        in_specs=[
            pl.BlockSpec((_BLK, 2 * _D), lambda b: (b, 0)),
            pl.BlockSpec((_BLK, 2 * _D), lambda b: (b, 0)),
            pl.BlockSpec((_BLK, 1), lambda b: (b, 0)),
            pl.BlockSpec((_BLK, 1), lambda b: (b, 0)),
            pl.BlockSpec((_D, 128), lambda b: (0, 0)),
            pl.BlockSpec((_D, 128), lambda b: (0, 0)),
            pl.BlockSpec((1, 128), lambda b: (0, 0)),
            pl.BlockSpec((128, _D), lambda b: (0, 0)),
            pl.BlockSpec((1, _D), lambda b: (0, 0)),
            pl.BlockSpec((1, _D), lambda b: (0, 0)),
            pl.BlockSpec((1, 1), lambda b: (0, 0)),
        ],
        out_specs=pl.BlockSpec((_BLK, 1), lambda b: (b, 0)),
        out_shape=jax.ShapeDtypeStruct((_BATCH, 1), jnp.float32),
    )(uo, io, ub, ib, w1u, w1i, b1, w2, b2, w3, b3)


def kernel(user_index, item_index, user_emb, item_emb, W1, b1, W2, b2, W3, b3):
    uidx = user_index.astype(jnp.int32)
    iidx = item_index.astype(jnp.int32)
    uo, io = _sc_gather(uidx >> 1, iidx >> 1,
                        user_emb.reshape(-1, 2 * _D),
                        item_emb.reshape(-1, 2 * _D))
    ub = (uidx & 1).reshape(_BATCH, 1)
    ib = (iidx & 1).reshape(_BATCH, 1)
    out2d = _mlp(uo, io, ub, ib,
                 W1[:, :_D].T, W1[:, _D:].T, b1.reshape(1, 128),
                 W2.T, b2.reshape(1, _D), W3.reshape(1, _D),
                 b3.reshape(1, 1))
    return out2d.reshape(_BATCH)
